# baseline (speedup 1.0000x reference)
import jax
import jax.numpy as jnp
import numpy as np
from jax.experimental import pallas as pl

NODE_TYPES = ["user", "item"]
EDGE_KEYS = [("user", "to", "item"), ("item", "to", "user")]
N = 10000; DIN = 128; HID = 128; H = 8; DH = 16; E = 160000; DE = 16; L = 2


def _noop_pallas(x):
    def body(x_ref, o_ref):
        o_ref[...] = x_ref[...]
    return pl.pallas_call(body, out_shape=jax.ShapeDtypeStruct(x.shape, x.dtype))(x)


def _ekey(et):
    return et[0] + "__" + et[1] + "__" + et[2]


def _arg_names():
    names = ["x_user", "x_item", "edge_index_user__to__item", "edge_attr_user__to__item", "edge_index_item__to__user", "edge_attr_item__to__user", "W_in_user", "b_in_user", "W_in_item", "b_in_item"]
    for l in range(L):
        for t in NODE_TYPES:
            for nm in ["Wq", "Wk", "Wv", "Wa"]:
                names.append(f"L{l}_{t}_{nm}")
            for nm in ["bq", "bk", "bv", "ba"]:
                names.append(f"L{l}_{t}_{nm}")
        for et in EDGE_KEYS:
            r = _ekey(et)
            for nm in ["Watt", "Wmsg", "prel", "We", "be"]:
                names.append(f"L{l}_{r}_{nm}")
    names.append("prelu_w")
    return names


def _hgt_layer(h, ei, ea, p, l):
    q = {}; k = {}; v = {}
    for t in NODE_TYPES:
        q[t] = (h[t] @ p[f"L{l}_{t}_Wq"] + p[f"L{l}_{t}_bq"]).reshape(-1, H, DH)
        k[t] = (h[t] @ p[f"L{l}_{t}_Wk"] + p[f"L{l}_{t}_bk"]).reshape(-1, H, DH)
        v[t] = (h[t] @ p[f"L{l}_{t}_Wv"] + p[f"L{l}_{t}_bv"]).reshape(-1, H, DH)
    agg = {t: jnp.zeros((h[t].shape[0], H, DH), jnp.float32) for t in NODE_TYPES}
    for et in EDGE_KEYS:
        src, _, dst = et
        r = _ekey(et)
        sidx = ei[r][0]; didx = ei[r][1]
        k_r = jnp.einsum('nhd,hde->nhe', k[src], p[f"L{l}_{r}_Watt"])
        v_r = jnp.einsum('nhd,hde->nhe', v[src], p[f"L{l}_{r}_Wmsg"])
        e_r = (ea[r] @ p[f"L{l}_{r}_We"] + p[f"L{l}_{r}_be"]).reshape(-1, H, DH)
        q_e = q[dst][didx]
        k_e = k_r[sidx] + e_r
        logits = (q_e * k_e).sum(-1) * p[f"L{l}_{r}_prel"] / np.sqrt(DH)
        nd = h[dst].shape[0]
        m = jax.ops.segment_max(logits, didx, num_segments=nd)
        m = jnp.where(jnp.isfinite(m), m, 0.0)
        ex = jnp.exp(logits - m[didx])
        den = jax.ops.segment_sum(ex, didx, num_segments=nd)
        alpha = ex / (den[didx] + 1e-16)
        msg = alpha[..., None] * (v_r[sidx] + e_r)
        agg[dst] = agg[dst] + jax.ops.segment_sum(msg, didx, num_segments=nd)
    out = {}
    for t in NODE_TYPES:
        a = jax.nn.gelu(agg[t].reshape(-1, HID))
        out[t] = a @ p[f"L{l}_{t}_Wa"] + p[f"L{l}_{t}_ba"] + h[t]
    return out


def kernel(*args):
    p = dict(zip(_arg_names(), args, strict=True))
    p["x_user"] = _noop_pallas(p["x_user"])
    h = {t: p["x_" + t] @ p["W_in_" + t] + p["b_in_" + t] for t in NODE_TYPES}
    ei = {_ekey(et): p["edge_index_" + _ekey(et)] for et in EDGE_KEYS}
    ea = {_ekey(et): p["edge_attr_" + _ekey(et)] for et in EDGE_KEYS}
    w = p["prelu_w"]
    for l in range(L):
        h = _hgt_layer(h, ei, ea, p, l)
        h = {t: jnp.where(h[t] > 0, h[t], w * h[t]) for t in NODE_TYPES}
    return h


# speedup vs baseline: 35.0754x; 35.0754x over previous
import jax
import jax.numpy as jnp
import numpy as np
from jax import lax
from jax.experimental import pallas as pl
from jax.experimental.pallas import tpu as pltpu
from jax.experimental.pallas import tpu_sc as plsc

_GDN = lax.GatherDimensionNumbers(
    offset_dims=(), collapsed_slice_dims=(0,), start_index_map=(0,))


def _permute(x, idx2d):
    return lax.gather(x, idx2d, _GDN, slice_sizes=(1,),
                      mode=lax.GatherScatterMode.PROMISE_IN_BOUNDS)


NODE_TYPES = ["user", "item"]
R_UI = "user__to__item"
R_IU = "item__to__user"
N = 10000
HID = 128
H = 8
DH = 16
E = 160000
L = 2

NS = 16
C = 80
EPW = E // NS
NCHUNK = EPW // C
HG = 64

F32 = jnp.float32


def _wprep_body(a_ref, m_ref, b_ref, wf_ref, bf_ref):
    m = m_ref[0]
    wf_ref[0] = jnp.dot(a_ref[0], m, preferred_element_type=F32)
    bf_ref[0] = jnp.dot(b_ref[0], m, preferred_element_type=F32)


def _wprep(a, m, b):
    B = a.shape[0]
    return pl.pallas_call(
        _wprep_body,
        grid=(B,),
        in_specs=[
            pl.BlockSpec((1, HID, HID), lambda i: (i, 0, 0)),
            pl.BlockSpec((1, HID, HID), lambda i: (i, 0, 0)),
            pl.BlockSpec((1, 1, HID), lambda i: (i, 0, 0)),
        ],
        out_specs=[
            pl.BlockSpec((1, HID, HID), lambda i: (i, 0, 0)),
            pl.BlockSpec((1, 1, HID), lambda i: (i, 0, 0)),
        ],
        out_shape=[
            jax.ShapeDtypeStruct((B, HID, HID), F32),
            jax.ShapeDtypeStruct((B, 1, HID), F32),
        ],
    )(a, m, b)


def _split_proj(r, q_ref, kvlo_ref, kvhi_ref):
    q_ref[...] = r[:, 0:HID]
    kvlo_ref[...] = jnp.concatenate([r[:, HID:HID + HG],
                                     r[:, 2 * HID:2 * HID + HG]], axis=1)
    kvhi_ref[...] = jnp.concatenate([r[:, HID + HG:2 * HID],
                                     r[:, 2 * HID + HG:3 * HID]], axis=1)


def _stage0_body(x_ref, w1_ref, b1_ref, w2_ref, b2_ref, h_ref,
                 q_ref, kvlo_ref, kvhi_ref):
    h = jnp.dot(x_ref[...], w1_ref[...], preferred_element_type=F32) + b1_ref[...]
    h_ref[...] = h
    r = jnp.dot(h, w2_ref[...], preferred_element_type=F32) + b2_ref[...]
    _split_proj(r, q_ref, kvlo_ref, kvhi_ref)


_PROJ_OUT_SPECS = [
    pl.BlockSpec((1000, HID), lambda i: (i, 0)),
    pl.BlockSpec((1000, 2 * HG), lambda i: (i, 0)),
    pl.BlockSpec((1000, 2 * HG), lambda i: (i, 0)),
]
_PROJ_OUT_SHAPE = [
    jax.ShapeDtypeStruct((N, HID), F32),
    jax.ShapeDtypeStruct((N, 2 * HG), F32),
    jax.ShapeDtypeStruct((N, 2 * HG), F32),
]


def _stage0(x, w_in, b_in, wcat, bcat, bm=1000):
    g = N // bm
    return pl.pallas_call(
        _stage0_body,
        grid=(g,),
        in_specs=[
            pl.BlockSpec((bm, HID), lambda i: (i, 0)),
            pl.BlockSpec((HID, HID), lambda i: (0, 0)),
            pl.BlockSpec((1, HID), lambda i: (0, 0)),
            pl.BlockSpec((HID, 3 * HID), lambda i: (0, 0)),
            pl.BlockSpec((1, 3 * HID), lambda i: (0, 0)),
        ],
        out_specs=[pl.BlockSpec((bm, HID), lambda i: (i, 0))] + _PROJ_OUT_SPECS,
        out_shape=[jax.ShapeDtypeStruct((N, HID), F32)] + _PROJ_OUT_SHAPE,
    )(x, w_in, b_in, wcat, bcat)


def _emm_body(x_ref, w_ref, b_ref, lo_ref, hi_ref):
    r = jnp.dot(x_ref[...], w_ref[...], preferred_element_type=F32) + b_ref[...]
    lo_ref[...] = r[:, :HG]
    hi_ref[...] = r[:, HG:]


def _emm(x, w, b, bm=2000):
    g = E // bm
    de = x.shape[1]
    return pl.pallas_call(
        _emm_body,
        grid=(g,),
        in_specs=[
            pl.BlockSpec((bm, de), lambda i: (i, 0)),
            pl.BlockSpec((de, HID), lambda i: (0, 0)),
            pl.BlockSpec((1, HID), lambda i: (0, 0)),
        ],
        out_specs=[
            pl.BlockSpec((bm, HG), lambda i: (i, 0)),
            pl.BlockSpec((bm, HG), lambda i: (i, 0)),
        ],
        out_shape=[
            jax.ShapeDtypeStruct((E, HG), F32),
            jax.ShapeDtypeStruct((E, HG), F32),
        ],
    )(x, w, b)


def _post_common(nlo_ref, nhi_ref, den_ref, s_ref, h_ref, wa_ref, ba_ref, pw_ref):
    den_b = jnp.dot(den_ref[...], s_ref[...], preferred_element_type=F32) + 1e-16
    num = jnp.concatenate([nlo_ref[...], nhi_ref[...]], axis=1)
    agg = num / den_b
    a = jax.nn.gelu(agg)
    hn = jnp.dot(a, wa_ref[...], preferred_element_type=F32) + ba_ref[...] + h_ref[...]
    pw = pw_ref[...]
    return jnp.where(hn > 0, hn, pw * hn)


def _post_body(nlo_ref, nhi_ref, den_ref, s_ref, h_ref, wa_ref, ba_ref, pw_ref,
               w2_ref, b2_ref, hn_ref, q_ref, kvlo_ref, kvhi_ref):
    hn = _post_common(nlo_ref, nhi_ref, den_ref, s_ref, h_ref, wa_ref, ba_ref,
                      pw_ref)
    hn_ref[...] = hn
    r = jnp.dot(hn, w2_ref[...], preferred_element_type=F32) + b2_ref[...]
    _split_proj(r, q_ref, kvlo_ref, kvhi_ref)


_POST_IN_SPECS = [
    pl.BlockSpec((1000, HG), lambda i: (i, 0)),
    pl.BlockSpec((1000, HG), lambda i: (i, 0)),
    pl.BlockSpec((1000, 16), lambda i: (i, 0)),
    pl.BlockSpec((16, HID), lambda i: (0, 0)),
    pl.BlockSpec((1000, HID), lambda i: (i, 0)),
    pl.BlockSpec((HID, HID), lambda i: (0, 0)),
    pl.BlockSpec((1, HID), lambda i: (0, 0)),
    pl.BlockSpec((1, HID), lambda i: (0, 0)),
]


def _post_proj(nlo, nhi, den, sel, h, wa, ba, pw, wcat, bcat, bm=1000):
    g = N // bm
    return pl.pallas_call(
        _post_body,
        grid=(g,),
        in_specs=_POST_IN_SPECS + [
            pl.BlockSpec((HID, 3 * HID), lambda i: (0, 0)),
            pl.BlockSpec((1, 3 * HID), lambda i: (0, 0)),
        ],
        out_specs=[pl.BlockSpec((bm, HID), lambda i: (i, 0))] + _PROJ_OUT_SPECS,
        out_shape=[jax.ShapeDtypeStruct((N, HID), F32)] + _PROJ_OUT_SHAPE,
    )(nlo, nhi, den, sel, h, wa, ba, pw, wcat, bcat)


def _final_body(nlo_ref, nhi_ref, den_ref, s_ref, h_ref, wa_ref, ba_ref,
                pw_ref, hn_ref):
    hn_ref[...] = _post_common(nlo_ref, nhi_ref, den_ref, s_ref, h_ref, wa_ref,
                               ba_ref, pw_ref)


def _post_final(nlo, nhi, den, sel, h, wa, ba, pw, bm=1000):
    g = N // bm
    return pl.pallas_call(
        _final_body,
        grid=(g,),
        in_specs=_POST_IN_SPECS,
        out_specs=pl.BlockSpec((bm, HID), lambda i: (i, 0)),
        out_shape=jax.ShapeDtypeStruct((N, HID), F32),
    )(nlo, nhi, den, sel, h, wa, ba, pw)


def _sc_body(d_ui, s_ui, q_i, kvlo_u, kvhi_u, elo_ui, ehi_ui,
             d_iu, s_iu, q_u, kvlo_i, kvhi_i, elo_iu, ehi_iu,
             nlo_i_o, nhi_i_o, den_i_o, nlo_u_o, nhi_u_o, den_u_o,
             didx_v, sidx_v, q_v, kv_v, e_v, msg_v, dex_v,
             zrow_v, zden_v, rowidx_v, num_sp, den_sp, sq, skv, se):
    c = lax.axis_index("c")
    s = lax.axis_index("s")
    lane = lax.iota(jnp.int32, 16)
    zero16 = jnp.zeros((16,), F32)
    perms = [jnp.bitwise_and(lane + sh, 15)[:, None] for sh in (8, 4, 2, 1)]

    row0 = s * 640

    def _zrow(i, _):
        for g in range(HG // 16):
            zrow_v[i, pl.ds(16 * g, 16)] = zero16
        return 0
    lax.fori_loop(0, 80, _zrow, 0)

    def _zden(i, _):
        zden_v[i, :] = zero16
        return 0
    lax.fori_loop(0, 80, _zden, 0)

    def _fill_rowidx(b):
        base = row0 + b * 80
        for g in range(5):
            rowidx_v[pl.ds(16 * g, 16)] = (
                jnp.full((16,), base + 16 * g, jnp.int32) + lane)

    def _for_blocks(fn):
        for b in range(8):
            @pl.when((s < 15) | (b < 5))
            def _():
                _fill_rowidx(b)
                fn(b)

    def _zero_acc(b):
        pltpu.sync_copy(zrow_v, num_sp.at[rowidx_v])
        pltpu.sync_copy(zden_v, den_sp.at[rowidx_v])

    def _zero_num(b):
        pltpu.sync_copy(zrow_v, num_sp.at[rowidx_v])

    def _run_pass(didx_h, sidx_h, q_h, kv_h, er_h, hbase):
        def _chunk(j, _):
            base = s * EPW + j * C
            pltpu.sync_copy(didx_h.at[pl.ds(base, C)], didx_v)
            pltpu.sync_copy(sidx_h.at[pl.ds(base, C)], sidx_v)
            cq = pltpu.async_copy(q_h.at[didx_v], q_v, sq)
            ckv = pltpu.async_copy(kv_h.at[sidx_v], kv_v, skv)
            ce = pltpu.async_copy(er_h.at[pl.ds(base, C)], e_v, se)
            cq.wait()
            ckv.wait()
            ce.wait()

            def _edge(i, _):
                dex = zero16
                for hh in range(4):
                    sl = pl.ds(16 * hh, 16)
                    qh = q_v[i, pl.ds(16 * (hbase + hh), 16)]
                    kh = kv_v[i, sl]
                    eh = e_v[i, sl]
                    ssum = qh * (kh + eh)
                    for perm in perms:
                        ssum = ssum + _permute(ssum, perm)
                    exh = jnp.exp(ssum)
                    vh = kv_v[i, pl.ds(HG + 16 * hh, 16)]
                    msg_v[i, sl] = exh * (vh + eh)
                    dex = jnp.where(lane == hbase + hh, exh, dex)
                dex_v[i, :] = dex
                return 0
            lax.fori_loop(0, C, _edge, 0)

            pltpu.sync_copy(msg_v, num_sp.at[didx_v], add=True)
            pltpu.sync_copy(dex_v, den_sp.at[didx_v], add=True)
            return 0
        lax.fori_loop(0, NCHUNK, _chunk, 0)

    def _writeout(sp, out, bounce):
        def _blk(b):
            pltpu.sync_copy(sp.at[rowidx_v], bounce)
            pltpu.sync_copy(bounce, out.at[pl.ds(row0 + b * 80, 80)])
        return _blk

    def _run_relation(didx_h, sidx_h, q_h, kvlo, kvhi, elo, ehi,
                      nlo_o, nhi_o, den_o):
        _for_blocks(_zero_acc)
        plsc.subcore_barrier()
        _run_pass(didx_h, sidx_h, q_h, kvlo, elo, 0)
        plsc.subcore_barrier()
        _for_blocks(_writeout(num_sp, nlo_o, msg_v))
        _for_blocks(_zero_num)
        plsc.subcore_barrier()
        _run_pass(didx_h, sidx_h, q_h, kvhi, ehi, 4)
        plsc.subcore_barrier()
        _for_blocks(_writeout(num_sp, nhi_o, msg_v))
        _for_blocks(_writeout(den_sp, den_o, dex_v))

    @pl.when(c == 0)
    def _():
        _run_relation(d_ui, s_ui, q_i, kvlo_u, kvhi_u, elo_ui, ehi_ui,
                      nlo_i_o, nhi_i_o, den_i_o)

    @pl.when(c == 1)
    def _():
        _run_relation(d_iu, s_iu, q_u, kvlo_i, kvhi_i, elo_iu, ehi_iu,
                      nlo_u_o, nhi_u_o, den_u_o)


def _sc_layer(d_ui, s_ui, q_item, kv_user, er_ui,
              d_iu, s_iu, q_user, kv_item, er_iu):
    mesh = plsc.VectorSubcoreMesh(core_axis_name="c", subcore_axis_name="s",
                                  num_cores=2, num_subcores=NS)
    return pl.kernel(
        _sc_body,
        out_type=[
            jax.ShapeDtypeStruct((N, HG), F32),
            jax.ShapeDtypeStruct((N, HG), F32),
            jax.ShapeDtypeStruct((N, 16), F32),
            jax.ShapeDtypeStruct((N, HG), F32),
            jax.ShapeDtypeStruct((N, HG), F32),
            jax.ShapeDtypeStruct((N, 16), F32),
        ],
        mesh=mesh,
        scratch_types=[
            pltpu.VMEM((C,), jnp.int32),
            pltpu.VMEM((C,), jnp.int32),
            pltpu.VMEM((C, HID), F32),
            pltpu.VMEM((C, 2 * HG), F32),
            pltpu.VMEM((C, HG), F32),
            pltpu.VMEM((C, HG), F32),
            pltpu.VMEM((C, 16), F32),
            pltpu.VMEM((80, HG), F32),
            pltpu.VMEM((80, 16), F32),
            pltpu.VMEM((80,), jnp.int32),
            pltpu.VMEM_SHARED((N, HG), F32),
            pltpu.VMEM_SHARED((N, 16), F32),
            pltpu.SemaphoreType.DMA,
            pltpu.SemaphoreType.DMA,
            pltpu.SemaphoreType.DMA,
        ],
    )(d_ui, s_ui, q_item, *kv_user, *er_ui,
      d_iu, s_iu, q_user, *kv_item, *er_iu)


def _arg_names():
    names = ["x_user", "x_item",
             "edge_index_user__to__item", "edge_attr_user__to__item",
             "edge_index_item__to__user", "edge_attr_item__to__user",
             "W_in_user", "b_in_user", "W_in_item", "b_in_item"]
    for l in range(L):
        for t in NODE_TYPES:
            for nm in ["Wq", "Wk", "Wv", "Wa"]:
                names.append(f"L{l}_{t}_{nm}")
            for nm in ["bq", "bk", "bv", "ba"]:
                names.append(f"L{l}_{t}_{nm}")
        for r in (R_UI, R_IU):
            for nm in ["Watt", "Wmsg", "prel", "We", "be"]:
                names.append(f"L{l}_{r}_{nm}")
    names.append("prelu_w")
    return names


def _blockdiag(w):
    eye = jnp.eye(H, dtype=w.dtype)
    return (eye[:, None, :, None] * w[:, :, None, :]).reshape(HID, HID)


def kernel(*args):
    p = dict(zip(_arg_names(), args, strict=True))

    q_rel = {"user": R_IU, "item": R_UI}
    src_rel = {"user": R_UI, "item": R_IU}

    a_stack, m_stack, b_stack = [], [], []
    for l in range(L):
        for t in NODE_TYPES:
            scale = jnp.repeat(p[f"L{l}_{q_rel[t]}_prel"], DH) * (1.0 / np.sqrt(DH))
            m_stack += [jnp.diag(scale.astype(F32)),
                        _blockdiag(p[f"L{l}_{src_rel[t]}_Watt"]),
                        _blockdiag(p[f"L{l}_{src_rel[t]}_Wmsg"])]
            a_stack += [p[f"L{l}_{t}_Wq"], p[f"L{l}_{t}_Wk"], p[f"L{l}_{t}_Wv"]]
            b_stack += [p[f"L{l}_{t}_bq"].reshape(1, HID),
                        p[f"L{l}_{t}_bk"].reshape(1, HID),
                        p[f"L{l}_{t}_bv"].reshape(1, HID)]
    wf, bf = _wprep(jnp.stack(a_stack), jnp.stack(m_stack), jnp.stack(b_stack))

    wcat, bcat = {}, {}
    for l in range(L):
        for ti, t in enumerate(NODE_TYPES):
            m = l * 2 + ti
            wcat[(l, t)] = jnp.transpose(wf[3 * m:3 * m + 3], (1, 0, 2)).reshape(HID, 3 * HID)
            bcat[(l, t)] = jnp.transpose(bf[3 * m:3 * m + 3], (1, 0, 2)).reshape(1, 3 * HID)

    er = {}
    for l in range(L):
        for r in (R_UI, R_IU):
            er[(l, r)] = _emm(p[f"edge_attr_{r}"], p[f"L{l}_{r}_We"],
                              p[f"L{l}_{r}_be"].reshape(1, HID))

    sidx = {r: p[f"edge_index_{r}"][0].astype(jnp.int32) for r in (R_UI, R_IU)}
    didx = {r: p[f"edge_index_{r}"][1].astype(jnp.int32) for r in (R_UI, R_IU)}

    sel = (jnp.eye(16, 8, dtype=F32)[:, :, None] *
           jnp.ones((1, 1, DH), F32)).reshape(16, HID)

    h, q, kv = {}, {}, {}
    for t in NODE_TYPES:
        h[t], q[t], kvlo, kvhi = _stage0(
            p[f"x_{t}"], p[f"W_in_{t}"], p[f"b_in_{t}"].reshape(1, HID),
            wcat[(0, t)], bcat[(0, t)])
        kv[t] = (kvlo, kvhi)

    for l in range(L):
        nlo_i, nhi_i, den_i, nlo_u, nhi_u, den_u = _sc_layer(
            didx[R_UI], sidx[R_UI], q["item"], kv["user"], er[(l, R_UI)],
            didx[R_IU], sidx[R_IU], q["user"], kv["item"], er[(l, R_IU)])
        num = {"item": (nlo_i, nhi_i), "user": (nlo_u, nhi_u)}
        den = {"item": den_i, "user": den_u}
        pw = p["prelu_w"].reshape(1, HID)
        if l + 1 < L:
            for t in NODE_TYPES:
                h[t], q[t], kvlo, kvhi = _post_proj(
                    num[t][0], num[t][1], den[t], sel, h[t],
                    p[f"L{l}_{t}_Wa"], p[f"L{l}_{t}_ba"].reshape(1, HID), pw,
                    wcat[(l + 1, t)], bcat[(l + 1, t)])
                kv[t] = (kvlo, kvhi)
        else:
            for t in NODE_TYPES:
                h[t] = _post_final(
                    num[t][0], num[t][1], den[t], sel, h[t],
                    p[f"L{l}_{t}_Wa"], p[f"L{l}_{t}_ba"].reshape(1, HID), pw)
    return h


# speedup vs baseline: 44.2879x; 1.2626x over previous
import jax
import jax.numpy as jnp
import numpy as np
from jax import lax
from jax.experimental import pallas as pl
from jax.experimental.pallas import tpu as pltpu
from jax.experimental.pallas import tpu_sc as plsc

_GDN = lax.GatherDimensionNumbers(
    offset_dims=(), collapsed_slice_dims=(0,), start_index_map=(0,))


def _permute(x, idx2d):
    return lax.gather(x, idx2d, _GDN, slice_sizes=(1,),
                      mode=lax.GatherScatterMode.PROMISE_IN_BOUNDS)


NODE_TYPES = ["user", "item"]
R_UI = "user__to__item"
R_IU = "item__to__user"
N = 10000
HID = 128
H = 8
DH = 16
E = 160000
L = 2

NS = 16
C = 80
EPW = E // NS
NCHUNK = EPW // C
HG = 64
AW = HG + 16

F32 = jnp.float32


def _wprep_body(a_ref, m_ref, b_ref, wf_ref, bf_ref):
    m = m_ref[0]
    wf_ref[0] = jnp.dot(a_ref[0], m, preferred_element_type=F32)
    bf_ref[0] = jnp.dot(b_ref[0], m, preferred_element_type=F32)


def _wprep(a, m, b):
    B = a.shape[0]
    return pl.pallas_call(
        _wprep_body,
        grid=(B,),
        in_specs=[
            pl.BlockSpec((1, HID, HID), lambda i: (i, 0, 0)),
            pl.BlockSpec((1, HID, HID), lambda i: (i, 0, 0)),
            pl.BlockSpec((1, 1, HID), lambda i: (i, 0, 0)),
        ],
        out_specs=[
            pl.BlockSpec((1, HID, HID), lambda i: (i, 0, 0)),
            pl.BlockSpec((1, 1, HID), lambda i: (i, 0, 0)),
        ],
        out_shape=[
            jax.ShapeDtypeStruct((B, HID, HID), F32),
            jax.ShapeDtypeStruct((B, 1, HID), F32),
        ],
    )(a, m, b)


def _split_proj(r, q_ref, kvlo_ref, kvhi_ref):
    q_ref[...] = r[:, 0:HID]
    kvlo_ref[...] = jnp.concatenate([r[:, HID:HID + HG],
                                     r[:, 2 * HID:2 * HID + HG]], axis=1)
    kvhi_ref[...] = jnp.concatenate([r[:, HID + HG:2 * HID],
                                     r[:, 2 * HID + HG:3 * HID]], axis=1)


def _stage0_body(x_ref, w1_ref, b1_ref, w2_ref, b2_ref, h_ref,
                 q_ref, kvlo_ref, kvhi_ref):
    h = jnp.dot(x_ref[...], w1_ref[...], preferred_element_type=F32) + b1_ref[...]
    h_ref[...] = h
    r = jnp.dot(h, w2_ref[...], preferred_element_type=F32) + b2_ref[...]
    _split_proj(r, q_ref, kvlo_ref, kvhi_ref)


_PROJ_OUT_SPECS = [
    pl.BlockSpec((1000, HID), lambda i: (i, 0)),
    pl.BlockSpec((1000, 2 * HG), lambda i: (i, 0)),
    pl.BlockSpec((1000, 2 * HG), lambda i: (i, 0)),
]
_PROJ_OUT_SHAPE = [
    jax.ShapeDtypeStruct((N, HID), F32),
    jax.ShapeDtypeStruct((N, 2 * HG), F32),
    jax.ShapeDtypeStruct((N, 2 * HG), F32),
]


def _stage0(x, w_in, b_in, wcat, bcat, bm=1000):
    g = N // bm
    return pl.pallas_call(
        _stage0_body,
        grid=(g,),
        in_specs=[
            pl.BlockSpec((bm, HID), lambda i: (i, 0)),
            pl.BlockSpec((HID, HID), lambda i: (0, 0)),
            pl.BlockSpec((1, HID), lambda i: (0, 0)),
            pl.BlockSpec((HID, 3 * HID), lambda i: (0, 0)),
            pl.BlockSpec((1, 3 * HID), lambda i: (0, 0)),
        ],
        out_specs=[pl.BlockSpec((bm, HID), lambda i: (i, 0))] + _PROJ_OUT_SPECS,
        out_shape=[jax.ShapeDtypeStruct((N, HID), F32)] + _PROJ_OUT_SHAPE,
    )(x, w_in, b_in, wcat, bcat)


def _emm_body(x_ref, w_ref, b_ref, lo_ref, hi_ref):
    r = jnp.dot(x_ref[...], w_ref[...], preferred_element_type=F32) + b_ref[...]
    lo_ref[...] = r[:, :HG]
    hi_ref[...] = r[:, HG:]


def _emm(x, w, b, bm=2000):
    g = E // bm
    de = x.shape[1]
    return pl.pallas_call(
        _emm_body,
        grid=(g,),
        in_specs=[
            pl.BlockSpec((bm, de), lambda i: (i, 0)),
            pl.BlockSpec((de, HID), lambda i: (0, 0)),
            pl.BlockSpec((1, HID), lambda i: (0, 0)),
        ],
        out_specs=[
            pl.BlockSpec((bm, HG), lambda i: (i, 0)),
            pl.BlockSpec((bm, HG), lambda i: (i, 0)),
        ],
        out_shape=[
            jax.ShapeDtypeStruct((E, HG), F32),
            jax.ShapeDtypeStruct((E, HG), F32),
        ],
    )(x, w, b)


def _post_common(nlo_ref, nhi_ref, s_ref, h_ref, wa_ref, ba_ref, pw_ref):
    nlo = nlo_ref[...]
    nhi = nhi_ref[...]
    den16 = nlo[:, HG:] + nhi[:, HG:]
    den_b = jnp.dot(den16, s_ref[...], preferred_element_type=F32) + 1e-16
    num = jnp.concatenate([nlo[:, :HG], nhi[:, :HG]], axis=1)
    agg = num / den_b
    a = jax.nn.gelu(agg)
    hn = jnp.dot(a, wa_ref[...], preferred_element_type=F32) + ba_ref[...] + h_ref[...]
    pw = pw_ref[...]
    return jnp.where(hn > 0, hn, pw * hn)


def _post_body(nlo_ref, nhi_ref, s_ref, h_ref, wa_ref, ba_ref, pw_ref,
               w2_ref, b2_ref, hn_ref, q_ref, kvlo_ref, kvhi_ref):
    hn = _post_common(nlo_ref, nhi_ref, s_ref, h_ref, wa_ref, ba_ref,
                      pw_ref)
    hn_ref[...] = hn
    r = jnp.dot(hn, w2_ref[...], preferred_element_type=F32) + b2_ref[...]
    _split_proj(r, q_ref, kvlo_ref, kvhi_ref)


_POST_IN_SPECS = [
    pl.BlockSpec((1000, HG + 16), lambda i: (i, 0)),
    pl.BlockSpec((1000, HG + 16), lambda i: (i, 0)),
    pl.BlockSpec((16, HID), lambda i: (0, 0)),
    pl.BlockSpec((1000, HID), lambda i: (i, 0)),
    pl.BlockSpec((HID, HID), lambda i: (0, 0)),
    pl.BlockSpec((1, HID), lambda i: (0, 0)),
    pl.BlockSpec((1, HID), lambda i: (0, 0)),
]


def _post_proj(nlo, nhi, sel, h, wa, ba, pw, wcat, bcat, bm=1000):
    g = N // bm
    return pl.pallas_call(
        _post_body,
        grid=(g,),
        in_specs=_POST_IN_SPECS + [
            pl.BlockSpec((HID, 3 * HID), lambda i: (0, 0)),
            pl.BlockSpec((1, 3 * HID), lambda i: (0, 0)),
        ],
        out_specs=[pl.BlockSpec((bm, HID), lambda i: (i, 0))] + _PROJ_OUT_SPECS,
        out_shape=[jax.ShapeDtypeStruct((N, HID), F32)] + _PROJ_OUT_SHAPE,
    )(nlo, nhi, sel, h, wa, ba, pw, wcat, bcat)


def _final_body(nlo_ref, nhi_ref, s_ref, h_ref, wa_ref, ba_ref,
                pw_ref, hn_ref):
    hn_ref[...] = _post_common(nlo_ref, nhi_ref, s_ref, h_ref, wa_ref,
                               ba_ref, pw_ref)


def _post_final(nlo, nhi, sel, h, wa, ba, pw, bm=1000):
    g = N // bm
    return pl.pallas_call(
        _final_body,
        grid=(g,),
        in_specs=_POST_IN_SPECS,
        out_specs=pl.BlockSpec((bm, HID), lambda i: (i, 0)),
        out_shape=jax.ShapeDtypeStruct((N, HID), F32),
    )(nlo, nhi, sel, h, wa, ba, pw)


def _sc_body(d_ui, s_ui, q_i, kvlo_u, kvhi_u, elo_ui, ehi_ui,
             d_iu, s_iu, q_u, kvlo_i, kvhi_i, elo_iu, ehi_iu,
             nlo_i_o, nhi_i_o, nlo_u_o, nhi_u_o,
             didx_v, sidx_v, q_v, kv_v, e_v,
             didx2_v, q2_v, kv2_v, e2_v, msg_v, num_sp,
             sq, skv, se, sq2, skv2, se2):
    c = lax.axis_index("c")
    s = lax.axis_index("s")
    lane = lax.iota(jnp.int32, 16)
    zero16 = jnp.zeros((16,), F32)
    perms = [jnp.bitwise_and(lane + sh, 15)[:, None] for sh in (8, 4, 2, 1)]

    row0 = s * 640

    def _zero_msg():
        def _zr(i, _):
            for g in range(AW // 16):
                msg_v[i, pl.ds(16 * g, 16)] = zero16
            return 0
        lax.fori_loop(0, 80, _zr, 0)

    def _fill_rowidx(b):
        base = row0 + b * 80
        for g in range(5):
            didx_v[pl.ds(16 * g, 16)] = (
                jnp.full((16,), base + 16 * g, jnp.int32) + lane)

    def _for_blocks(fn):
        for b in range(8):
            @pl.when((s < 15) | (b < 5))
            def _():
                _fill_rowidx(b)
                fn(b)

    def _zero_num(b):
        pltpu.sync_copy(msg_v, num_sp.at[didx_v])

    def _run_pass(didx_h, sidx_h, q_h, kv_h, er_h, hbase):
        bufs = ((didx_v, q_v, kv_v, e_v, sq, skv, se),
                (didx2_v, q2_v, kv2_v, e2_v, sq2, skv2, se2))

        def _fire(j, bi):
            d, qv, kvv, ev, s1, s2, s3 = bufs[bi]
            base = s * EPW + j * C
            pltpu.sync_copy(didx_h.at[pl.ds(base, C)], d)
            pltpu.sync_copy(sidx_h.at[pl.ds(base, C)], sidx_v)
            pltpu.async_copy(q_h.at[d], qv, s1)
            pltpu.async_copy(kv_h.at[sidx_v], kvv, s2)
            pltpu.async_copy(er_h.at[pl.ds(base, C)], ev, s3)

        def _wait(bi):
            d, qv, kvv, ev, s1, s2, s3 = bufs[bi]
            pltpu.make_async_copy(q_h.at[d], qv, s1).wait()
            pltpu.make_async_copy(kv_h.at[sidx_v], kvv, s2).wait()
            pltpu.make_async_copy(er_h.at[pl.ds(0, C)], ev, s3).wait()

        def _consume(bi):
            d, qv, kvv, ev, s1, s2, s3 = bufs[bi]

            def _edge(i, _):
                dex = zero16
                for hh in range(4):
                    sl = pl.ds(16 * hh, 16)
                    qh = qv[i, pl.ds(16 * (hbase + hh), 16)]
                    kh = kvv[i, sl]
                    eh = ev[i, sl]
                    ssum = qh * (kh + eh)
                    for perm in perms:
                        ssum = ssum + _permute(ssum, perm)
                    exh = jnp.exp(ssum)
                    vh = kvv[i, pl.ds(HG + 16 * hh, 16)]
                    msg_v[i, sl] = exh * (vh + eh)
                    dex = jnp.where(lane == hbase + hh, exh, dex)
                msg_v[i, pl.ds(HG, 16)] = dex
                return 0
            lax.fori_loop(0, C, _edge, 0)

            pltpu.sync_copy(msg_v, num_sp.at[d], add=True)

        _fire(0, 0)

        def _pair(m, _):
            j0 = 2 * m
            _wait(0)

            @pl.when(j0 + 1 < NCHUNK)
            def _():
                _fire(j0 + 1, 1)
            _consume(0)

            @pl.when(j0 + 1 < NCHUNK)
            def _():
                _wait(1)

                @pl.when(j0 + 2 < NCHUNK)
                def _():
                    _fire(j0 + 2, 0)
                _consume(1)
            return 0
        lax.fori_loop(0, (NCHUNK + 1) // 2, _pair, 0)

    def _writeout(sp, out, bounce):
        def _blk(b):
            pltpu.sync_copy(sp.at[didx_v], bounce)
            pltpu.sync_copy(bounce, out.at[pl.ds(row0 + b * 80, 80)])
        return _blk

    def _run_relation(didx_h, sidx_h, q_h, kvlo, kvhi, elo, ehi,
                      nlo_o, nhi_o):
        _zero_msg()
        _for_blocks(_zero_num)
        plsc.subcore_barrier()
        _run_pass(didx_h, sidx_h, q_h, kvlo, elo, 0)
        plsc.subcore_barrier()
        _for_blocks(_writeout(num_sp, nlo_o, msg_v))
        _zero_msg()
        _for_blocks(_zero_num)
        plsc.subcore_barrier()
        _run_pass(didx_h, sidx_h, q_h, kvhi, ehi, 4)
        plsc.subcore_barrier()
        _for_blocks(_writeout(num_sp, nhi_o, msg_v))

    @pl.when(c == 0)
    def _():
        _run_relation(d_ui, s_ui, q_i, kvlo_u, kvhi_u, elo_ui, ehi_ui,
                      nlo_i_o, nhi_i_o)

    @pl.when(c == 1)
    def _():
        _run_relation(d_iu, s_iu, q_u, kvlo_i, kvhi_i, elo_iu, ehi_iu,
                      nlo_u_o, nhi_u_o)


def _sc_layer(d_ui, s_ui, q_item, kv_user, er_ui,
              d_iu, s_iu, q_user, kv_item, er_iu):
    mesh = plsc.VectorSubcoreMesh(core_axis_name="c", subcore_axis_name="s",
                                  num_cores=2, num_subcores=NS)
    return pl.kernel(
        _sc_body,
        out_type=[
            jax.ShapeDtypeStruct((N, AW), F32),
            jax.ShapeDtypeStruct((N, AW), F32),
            jax.ShapeDtypeStruct((N, AW), F32),
            jax.ShapeDtypeStruct((N, AW), F32),
        ],
        mesh=mesh,
        scratch_types=[
            pltpu.VMEM((C,), jnp.int32),
            pltpu.VMEM((C,), jnp.int32),
            pltpu.VMEM((C, HID), F32),
            pltpu.VMEM((C, 2 * HG), F32),
            pltpu.VMEM((C, HG), F32),
            pltpu.VMEM((C,), jnp.int32),
            pltpu.VMEM((C, HID), F32),
            pltpu.VMEM((C, 2 * HG), F32),
            pltpu.VMEM((C, HG), F32),
            pltpu.VMEM((C, AW), F32),
            pltpu.VMEM_SHARED((N, AW), F32),
            pltpu.SemaphoreType.DMA,
            pltpu.SemaphoreType.DMA,
            pltpu.SemaphoreType.DMA,
            pltpu.SemaphoreType.DMA,
            pltpu.SemaphoreType.DMA,
            pltpu.SemaphoreType.DMA,
        ],
    )(d_ui, s_ui, q_item, *kv_user, *er_ui,
      d_iu, s_iu, q_user, *kv_item, *er_iu)


def _arg_names():
    names = ["x_user", "x_item",
             "edge_index_user__to__item", "edge_attr_user__to__item",
             "edge_index_item__to__user", "edge_attr_item__to__user",
             "W_in_user", "b_in_user", "W_in_item", "b_in_item"]
    for l in range(L):
        for t in NODE_TYPES:
            for nm in ["Wq", "Wk", "Wv", "Wa"]:
                names.append(f"L{l}_{t}_{nm}")
            for nm in ["bq", "bk", "bv", "ba"]:
                names.append(f"L{l}_{t}_{nm}")
        for r in (R_UI, R_IU):
            for nm in ["Watt", "Wmsg", "prel", "We", "be"]:
                names.append(f"L{l}_{r}_{nm}")
    names.append("prelu_w")
    return names


def _blockdiag(w):
    eye = jnp.eye(H, dtype=w.dtype)
    return (eye[:, None, :, None] * w[:, :, None, :]).reshape(HID, HID)


def kernel(*args):
    p = dict(zip(_arg_names(), args, strict=True))

    q_rel = {"user": R_IU, "item": R_UI}
    src_rel = {"user": R_UI, "item": R_IU}

    a_stack, m_stack, b_stack = [], [], []
    for l in range(L):
        for t in NODE_TYPES:
            scale = jnp.repeat(p[f"L{l}_{q_rel[t]}_prel"], DH) * (1.0 / np.sqrt(DH))
            m_stack += [jnp.diag(scale.astype(F32)),
                        _blockdiag(p[f"L{l}_{src_rel[t]}_Watt"]),
                        _blockdiag(p[f"L{l}_{src_rel[t]}_Wmsg"])]
            a_stack += [p[f"L{l}_{t}_Wq"], p[f"L{l}_{t}_Wk"], p[f"L{l}_{t}_Wv"]]
            b_stack += [p[f"L{l}_{t}_bq"].reshape(1, HID),
                        p[f"L{l}_{t}_bk"].reshape(1, HID),
                        p[f"L{l}_{t}_bv"].reshape(1, HID)]
    wf, bf = _wprep(jnp.stack(a_stack), jnp.stack(m_stack), jnp.stack(b_stack))

    wcat, bcat = {}, {}
    for l in range(L):
        for ti, t in enumerate(NODE_TYPES):
            m = l * 2 + ti
            wcat[(l, t)] = jnp.transpose(wf[3 * m:3 * m + 3], (1, 0, 2)).reshape(HID, 3 * HID)
            bcat[(l, t)] = jnp.transpose(bf[3 * m:3 * m + 3], (1, 0, 2)).reshape(1, 3 * HID)

    er = {}
    for l in range(L):
        for r in (R_UI, R_IU):
            er[(l, r)] = _emm(p[f"edge_attr_{r}"], p[f"L{l}_{r}_We"],
                              p[f"L{l}_{r}_be"].reshape(1, HID))

    sidx = {r: p[f"edge_index_{r}"][0].astype(jnp.int32) for r in (R_UI, R_IU)}
    didx = {r: p[f"edge_index_{r}"][1].astype(jnp.int32) for r in (R_UI, R_IU)}

    sel = (jnp.eye(16, 8, dtype=F32)[:, :, None] *
           jnp.ones((1, 1, DH), F32)).reshape(16, HID)

    h, q, kv = {}, {}, {}
    for t in NODE_TYPES:
        h[t], q[t], kvlo, kvhi = _stage0(
            p[f"x_{t}"], p[f"W_in_{t}"], p[f"b_in_{t}"].reshape(1, HID),
            wcat[(0, t)], bcat[(0, t)])
        kv[t] = (kvlo, kvhi)

    for l in range(L):
        nlo_i, nhi_i, nlo_u, nhi_u = _sc_layer(
            didx[R_UI], sidx[R_UI], q["item"], kv["user"], er[(l, R_UI)],
            didx[R_IU], sidx[R_IU], q["user"], kv["item"], er[(l, R_IU)])
        num = {"item": (nlo_i, nhi_i), "user": (nlo_u, nhi_u)}
        pw = p["prelu_w"].reshape(1, HID)
        if l + 1 < L:
            for t in NODE_TYPES:
                h[t], q[t], kvlo, kvhi = _post_proj(
                    num[t][0], num[t][1], sel, h[t],
                    p[f"L{l}_{t}_Wa"], p[f"L{l}_{t}_ba"].reshape(1, HID), pw,
                    wcat[(l + 1, t)], bcat[(l + 1, t)])
                kv[t] = (kvlo, kvhi)
        else:
            for t in NODE_TYPES:
                h[t] = _post_final(
                    num[t][0], num[t][1], sel, h[t],
                    p[f"L{l}_{t}_Wa"], p[f"L{l}_{t}_ba"].reshape(1, HID), pw)
    return h


# speedup vs baseline: 44.2881x; 1.0000x over previous
import jax
import jax.numpy as jnp
import numpy as np
from jax import lax
from jax.experimental import pallas as pl
from jax.experimental.pallas import tpu as pltpu
from jax.experimental.pallas import tpu_sc as plsc

_GDN = lax.GatherDimensionNumbers(
    offset_dims=(), collapsed_slice_dims=(0,), start_index_map=(0,))


def _permute(x, idx2d):
    return lax.gather(x, idx2d, _GDN, slice_sizes=(1,),
                      mode=lax.GatherScatterMode.PROMISE_IN_BOUNDS)


NODE_TYPES = ["user", "item"]
R_UI = "user__to__item"
R_IU = "item__to__user"
N = 10000
HID = 128
H = 8
DH = 16
E = 160000
L = 2

NS = 16
C = 80
EPW = E // NS
NCHUNK = EPW // C
HG = 64
AW = HG + 16

F32 = jnp.float32


def _wprep_body(a_ref, m_ref, b_ref, wf_ref, bf_ref):
    m = m_ref[0]
    wf_ref[0] = jnp.dot(a_ref[0], m, preferred_element_type=F32)
    bf_ref[0] = jnp.dot(b_ref[0], m, preferred_element_type=F32)


def _wprep(a, m, b):
    B = a.shape[0]
    return pl.pallas_call(
        _wprep_body,
        grid=(B,),
        in_specs=[
            pl.BlockSpec((1, HID, HID), lambda i: (i, 0, 0)),
            pl.BlockSpec((1, HID, HID), lambda i: (i, 0, 0)),
            pl.BlockSpec((1, 1, HID), lambda i: (i, 0, 0)),
        ],
        out_specs=[
            pl.BlockSpec((1, HID, HID), lambda i: (i, 0, 0)),
            pl.BlockSpec((1, 1, HID), lambda i: (i, 0, 0)),
        ],
        out_shape=[
            jax.ShapeDtypeStruct((B, HID, HID), F32),
            jax.ShapeDtypeStruct((B, 1, HID), F32),
        ],
    )(a, m, b)


def _split_proj(r, q_ref, kvlo_ref, kvhi_ref):
    q_ref[...] = r[:, 0:HID]
    kvlo_ref[...] = jnp.concatenate([r[:, HID:HID + HG],
                                     r[:, 2 * HID:2 * HID + HG]], axis=1)
    kvhi_ref[...] = jnp.concatenate([r[:, HID + HG:2 * HID],
                                     r[:, 2 * HID + HG:3 * HID]], axis=1)


def _stage0_body(x_ref, w1_ref, b1_ref, w2_ref, b2_ref, h_ref,
                 q_ref, kvlo_ref, kvhi_ref):
    h = jnp.dot(x_ref[...], w1_ref[...], preferred_element_type=F32) + b1_ref[...]
    h_ref[...] = h
    r = jnp.dot(h, w2_ref[...], preferred_element_type=F32) + b2_ref[...]
    _split_proj(r, q_ref, kvlo_ref, kvhi_ref)


_PROJ_OUT_SPECS = [
    pl.BlockSpec((1000, HID), lambda i: (i, 0)),
    pl.BlockSpec((1000, 2 * HG), lambda i: (i, 0)),
    pl.BlockSpec((1000, 2 * HG), lambda i: (i, 0)),
]
_PROJ_OUT_SHAPE = [
    jax.ShapeDtypeStruct((N, HID), F32),
    jax.ShapeDtypeStruct((N, 2 * HG), F32),
    jax.ShapeDtypeStruct((N, 2 * HG), F32),
]


def _stage0(x, w_in, b_in, wcat, bcat, bm=1000):
    g = N // bm
    return pl.pallas_call(
        _stage0_body,
        grid=(g,),
        in_specs=[
            pl.BlockSpec((bm, HID), lambda i: (i, 0)),
            pl.BlockSpec((HID, HID), lambda i: (0, 0)),
            pl.BlockSpec((1, HID), lambda i: (0, 0)),
            pl.BlockSpec((HID, 3 * HID), lambda i: (0, 0)),
            pl.BlockSpec((1, 3 * HID), lambda i: (0, 0)),
        ],
        out_specs=[pl.BlockSpec((bm, HID), lambda i: (i, 0))] + _PROJ_OUT_SPECS,
        out_shape=[jax.ShapeDtypeStruct((N, HID), F32)] + _PROJ_OUT_SHAPE,
    )(x, w_in, b_in, wcat, bcat)


def _emm_body(x_ref, w_ref, b_ref, lo_ref, hi_ref):
    r = jnp.dot(x_ref[...], w_ref[...], preferred_element_type=F32) + b_ref[...]
    lo_ref[...] = r[:, :HG]
    hi_ref[...] = r[:, HG:]


def _emm(x, w, b, bm=2000):
    g = E // bm
    de = x.shape[1]
    return pl.pallas_call(
        _emm_body,
        grid=(g,),
        in_specs=[
            pl.BlockSpec((bm, de), lambda i: (i, 0)),
            pl.BlockSpec((de, HID), lambda i: (0, 0)),
            pl.BlockSpec((1, HID), lambda i: (0, 0)),
        ],
        out_specs=[
            pl.BlockSpec((bm, HG), lambda i: (i, 0)),
            pl.BlockSpec((bm, HG), lambda i: (i, 0)),
        ],
        out_shape=[
            jax.ShapeDtypeStruct((E, HG), F32),
            jax.ShapeDtypeStruct((E, HG), F32),
        ],
    )(x, w, b)


def _post_common(nlo_ref, nhi_ref, s_ref, h_ref, wa_ref, ba_ref, pw_ref):
    nlo = nlo_ref[...]
    nhi = nhi_ref[...]
    den16 = nlo[:, HG:] + nhi[:, HG:]
    den_b = jnp.dot(den16, s_ref[...], preferred_element_type=F32) + 1e-16
    num = jnp.concatenate([nlo[:, :HG], nhi[:, :HG]], axis=1)
    agg = num / den_b
    a = jax.nn.gelu(agg)
    hn = jnp.dot(a, wa_ref[...], preferred_element_type=F32) + ba_ref[...] + h_ref[...]
    pw = pw_ref[...]
    return jnp.where(hn > 0, hn, pw * hn)


def _post_body(nlo_ref, nhi_ref, s_ref, h_ref, wa_ref, ba_ref, pw_ref,
               w2_ref, b2_ref, hn_ref, q_ref, kvlo_ref, kvhi_ref):
    hn = _post_common(nlo_ref, nhi_ref, s_ref, h_ref, wa_ref, ba_ref,
                      pw_ref)
    hn_ref[...] = hn
    r = jnp.dot(hn, w2_ref[...], preferred_element_type=F32) + b2_ref[...]
    _split_proj(r, q_ref, kvlo_ref, kvhi_ref)


_POST_IN_SPECS = [
    pl.BlockSpec((1000, HG + 16), lambda i: (i, 0)),
    pl.BlockSpec((1000, HG + 16), lambda i: (i, 0)),
    pl.BlockSpec((16, HID), lambda i: (0, 0)),
    pl.BlockSpec((1000, HID), lambda i: (i, 0)),
    pl.BlockSpec((HID, HID), lambda i: (0, 0)),
    pl.BlockSpec((1, HID), lambda i: (0, 0)),
    pl.BlockSpec((1, HID), lambda i: (0, 0)),
]


def _post_proj(nlo, nhi, sel, h, wa, ba, pw, wcat, bcat, bm=1000):
    g = N // bm
    return pl.pallas_call(
        _post_body,
        grid=(g,),
        in_specs=_POST_IN_SPECS + [
            pl.BlockSpec((HID, 3 * HID), lambda i: (0, 0)),
            pl.BlockSpec((1, 3 * HID), lambda i: (0, 0)),
        ],
        out_specs=[pl.BlockSpec((bm, HID), lambda i: (i, 0))] + _PROJ_OUT_SPECS,
        out_shape=[jax.ShapeDtypeStruct((N, HID), F32)] + _PROJ_OUT_SHAPE,
    )(nlo, nhi, sel, h, wa, ba, pw, wcat, bcat)


def _final_body(nlo_ref, nhi_ref, s_ref, h_ref, wa_ref, ba_ref,
                pw_ref, hn_ref):
    hn_ref[...] = _post_common(nlo_ref, nhi_ref, s_ref, h_ref, wa_ref,
                               ba_ref, pw_ref)


def _post_final(nlo, nhi, sel, h, wa, ba, pw, bm=1000):
    g = N // bm
    return pl.pallas_call(
        _final_body,
        grid=(g,),
        in_specs=_POST_IN_SPECS,
        out_specs=pl.BlockSpec((bm, HID), lambda i: (i, 0)),
        out_shape=jax.ShapeDtypeStruct((N, HID), F32),
    )(nlo, nhi, sel, h, wa, ba, pw)


def _sc_body(d_ui, s_ui, q_i, kvlo_u, kvhi_u, elo_ui, ehi_ui,
             d_iu, s_iu, q_u, kvlo_i, kvhi_i, elo_iu, ehi_iu,
             nlo_i_o, nhi_i_o, nlo_u_o, nhi_u_o,
             didx_v, sidx_v, q_v, kv_v, e_v,
             didx2_v, q2_v, kv2_v, e2_v, msg_v, num_sp,
             sq, skv, se, sq2, skv2, se2):
    c = lax.axis_index("c")
    s = lax.axis_index("s")
    lane = lax.iota(jnp.int32, 16)
    zero16 = jnp.zeros((16,), F32)
    perms = [jnp.bitwise_and(lane + sh, 15)[:, None] for sh in (8, 4, 2, 1)]

    row0 = s * 640

    def _zero_msg():
        def _zr(i, _):
            for g in range(AW // 16):
                msg_v[i, pl.ds(16 * g, 16)] = zero16
            return 0
        lax.fori_loop(0, 80, _zr, 0)

    def _fill_rowidx(b):
        base = row0 + b * 80
        for g in range(5):
            didx_v[pl.ds(16 * g, 16)] = (
                jnp.full((16,), base + 16 * g, jnp.int32) + lane)

    def _for_blocks(fn):
        for b in range(8):
            @pl.when((s < 15) | (b < 5))
            def _():
                _fill_rowidx(b)
                fn(b)

    def _zero_num(b):
        pltpu.sync_copy(msg_v, num_sp.at[didx_v])

    def _run_pass(didx_h, sidx_h, q_h, kv_h, er_h, hbase):
        bufs = ((didx_v, q_v, kv_v, e_v, sq, skv, se),
                (didx2_v, q2_v, kv2_v, e2_v, sq2, skv2, se2))

        def _fire(j, bi):
            d, qv, kvv, ev, s1, s2, s3 = bufs[bi]
            base = s * EPW + j * C
            pltpu.sync_copy(didx_h.at[pl.ds(base, C)], d)
            pltpu.sync_copy(sidx_h.at[pl.ds(base, C)], sidx_v)
            pltpu.async_copy(q_h.at[d], qv, s1)
            pltpu.async_copy(kv_h.at[sidx_v], kvv, s2)
            pltpu.async_copy(er_h.at[pl.ds(base, C)], ev, s3)

        def _wait(bi):
            d, qv, kvv, ev, s1, s2, s3 = bufs[bi]
            pltpu.make_async_copy(q_h.at[d], qv, s1).wait()
            pltpu.make_async_copy(kv_h.at[sidx_v], kvv, s2).wait()
            pltpu.make_async_copy(er_h.at[pl.ds(0, C)], ev, s3).wait()

        def _consume(bi):
            d, qv, kvv, ev, s1, s2, s3 = bufs[bi]

            def _one_edge(i):
                dex = zero16
                for hh in range(4):
                    sl = pl.ds(16 * hh, 16)
                    qh = qv[i, pl.ds(16 * (hbase + hh), 16)]
                    kh = kvv[i, sl]
                    eh = ev[i, sl]
                    ssum = qh * (kh + eh)
                    for perm in perms:
                        ssum = ssum + _permute(ssum, perm)
                    exh = jnp.exp(ssum)
                    vh = kvv[i, pl.ds(HG + 16 * hh, 16)]
                    msg_v[i, sl] = exh * (vh + eh)
                    dex = jnp.where(lane == hbase + hh, exh, dex)
                msg_v[i, pl.ds(HG, 16)] = dex

            def _edge2(i2, _):
                _one_edge(2 * i2)
                _one_edge(2 * i2 + 1)
                return 0
            lax.fori_loop(0, C // 2, _edge2, 0)

            pltpu.sync_copy(msg_v, num_sp.at[d], add=True)

        _fire(0, 0)

        def _pair(m, _):
            j0 = 2 * m
            _wait(0)

            @pl.when(j0 + 1 < NCHUNK)
            def _():
                _fire(j0 + 1, 1)
            _consume(0)

            @pl.when(j0 + 1 < NCHUNK)
            def _():
                _wait(1)

                @pl.when(j0 + 2 < NCHUNK)
                def _():
                    _fire(j0 + 2, 0)
                _consume(1)
            return 0
        lax.fori_loop(0, (NCHUNK + 1) // 2, _pair, 0)

    def _writeout(sp, out, bounce):
        def _blk(b):
            pltpu.sync_copy(sp.at[didx_v], bounce)
            pltpu.sync_copy(bounce, out.at[pl.ds(row0 + b * 80, 80)])
        return _blk

    def _run_relation(didx_h, sidx_h, q_h, kvlo, kvhi, elo, ehi,
                      nlo_o, nhi_o):
        _zero_msg()
        _for_blocks(_zero_num)
        plsc.subcore_barrier()
        _run_pass(didx_h, sidx_h, q_h, kvlo, elo, 0)
        plsc.subcore_barrier()
        _for_blocks(_writeout(num_sp, nlo_o, msg_v))
        _zero_msg()
        _for_blocks(_zero_num)
        plsc.subcore_barrier()
        _run_pass(didx_h, sidx_h, q_h, kvhi, ehi, 4)
        plsc.subcore_barrier()
        _for_blocks(_writeout(num_sp, nhi_o, msg_v))

    @pl.when(c == 0)
    def _():
        _run_relation(d_ui, s_ui, q_i, kvlo_u, kvhi_u, elo_ui, ehi_ui,
                      nlo_i_o, nhi_i_o)

    @pl.when(c == 1)
    def _():
        _run_relation(d_iu, s_iu, q_u, kvlo_i, kvhi_i, elo_iu, ehi_iu,
                      nlo_u_o, nhi_u_o)


def _sc_layer(d_ui, s_ui, q_item, kv_user, er_ui,
              d_iu, s_iu, q_user, kv_item, er_iu):
    mesh = plsc.VectorSubcoreMesh(core_axis_name="c", subcore_axis_name="s",
                                  num_cores=2, num_subcores=NS)
    return pl.kernel(
        _sc_body,
        out_type=[
            jax.ShapeDtypeStruct((N, AW), F32),
            jax.ShapeDtypeStruct((N, AW), F32),
            jax.ShapeDtypeStruct((N, AW), F32),
            jax.ShapeDtypeStruct((N, AW), F32),
        ],
        mesh=mesh,
        scratch_types=[
            pltpu.VMEM((C,), jnp.int32),
            pltpu.VMEM((C,), jnp.int32),
            pltpu.VMEM((C, HID), F32),
            pltpu.VMEM((C, 2 * HG), F32),
            pltpu.VMEM((C, HG), F32),
            pltpu.VMEM((C,), jnp.int32),
            pltpu.VMEM((C, HID), F32),
            pltpu.VMEM((C, 2 * HG), F32),
            pltpu.VMEM((C, HG), F32),
            pltpu.VMEM((C, AW), F32),
            pltpu.VMEM_SHARED((N, AW), F32),
            pltpu.SemaphoreType.DMA,
            pltpu.SemaphoreType.DMA,
            pltpu.SemaphoreType.DMA,
            pltpu.SemaphoreType.DMA,
            pltpu.SemaphoreType.DMA,
            pltpu.SemaphoreType.DMA,
        ],
    )(d_ui, s_ui, q_item, *kv_user, *er_ui,
      d_iu, s_iu, q_user, *kv_item, *er_iu)


def _arg_names():
    names = ["x_user", "x_item",
             "edge_index_user__to__item", "edge_attr_user__to__item",
             "edge_index_item__to__user", "edge_attr_item__to__user",
             "W_in_user", "b_in_user", "W_in_item", "b_in_item"]
    for l in range(L):
        for t in NODE_TYPES:
            for nm in ["Wq", "Wk", "Wv", "Wa"]:
                names.append(f"L{l}_{t}_{nm}")
            for nm in ["bq", "bk", "bv", "ba"]:
                names.append(f"L{l}_{t}_{nm}")
        for r in (R_UI, R_IU):
            for nm in ["Watt", "Wmsg", "prel", "We", "be"]:
                names.append(f"L{l}_{r}_{nm}")
    names.append("prelu_w")
    return names


def _blockdiag(w):
    eye = jnp.eye(H, dtype=w.dtype)
    return (eye[:, None, :, None] * w[:, :, None, :]).reshape(HID, HID)


def kernel(*args):
    p = dict(zip(_arg_names(), args, strict=True))

    q_rel = {"user": R_IU, "item": R_UI}
    src_rel = {"user": R_UI, "item": R_IU}

    a_stack, m_stack, b_stack = [], [], []
    for l in range(L):
        for t in NODE_TYPES:
            scale = jnp.repeat(p[f"L{l}_{q_rel[t]}_prel"], DH) * (1.0 / np.sqrt(DH))
            m_stack += [jnp.diag(scale.astype(F32)),
                        _blockdiag(p[f"L{l}_{src_rel[t]}_Watt"]),
                        _blockdiag(p[f"L{l}_{src_rel[t]}_Wmsg"])]
            a_stack += [p[f"L{l}_{t}_Wq"], p[f"L{l}_{t}_Wk"], p[f"L{l}_{t}_Wv"]]
            b_stack += [p[f"L{l}_{t}_bq"].reshape(1, HID),
                        p[f"L{l}_{t}_bk"].reshape(1, HID),
                        p[f"L{l}_{t}_bv"].reshape(1, HID)]
    wf, bf = _wprep(jnp.stack(a_stack), jnp.stack(m_stack), jnp.stack(b_stack))

    wcat, bcat = {}, {}
    for l in range(L):
        for ti, t in enumerate(NODE_TYPES):
            m = l * 2 + ti
            wcat[(l, t)] = jnp.transpose(wf[3 * m:3 * m + 3], (1, 0, 2)).reshape(HID, 3 * HID)
            bcat[(l, t)] = jnp.transpose(bf[3 * m:3 * m + 3], (1, 0, 2)).reshape(1, 3 * HID)

    er = {}
    for l in range(L):
        for r in (R_UI, R_IU):
            er[(l, r)] = _emm(p[f"edge_attr_{r}"], p[f"L{l}_{r}_We"],
                              p[f"L{l}_{r}_be"].reshape(1, HID))

    sidx = {r: p[f"edge_index_{r}"][0].astype(jnp.int32) for r in (R_UI, R_IU)}
    didx = {r: p[f"edge_index_{r}"][1].astype(jnp.int32) for r in (R_UI, R_IU)}

    sel = (jnp.eye(16, 8, dtype=F32)[:, :, None] *
           jnp.ones((1, 1, DH), F32)).reshape(16, HID)

    h, q, kv = {}, {}, {}
    for t in NODE_TYPES:
        h[t], q[t], kvlo, kvhi = _stage0(
            p[f"x_{t}"], p[f"W_in_{t}"], p[f"b_in_{t}"].reshape(1, HID),
            wcat[(0, t)], bcat[(0, t)])
        kv[t] = (kvlo, kvhi)

    for l in range(L):
        nlo_i, nhi_i, nlo_u, nhi_u = _sc_layer(
            didx[R_UI], sidx[R_UI], q["item"], kv["user"], er[(l, R_UI)],
            didx[R_IU], sidx[R_IU], q["user"], kv["item"], er[(l, R_IU)])
        num = {"item": (nlo_i, nhi_i), "user": (nlo_u, nhi_u)}
        pw = p["prelu_w"].reshape(1, HID)
        if l + 1 < L:
            for t in NODE_TYPES:
                h[t], q[t], kvlo, kvhi = _post_proj(
                    num[t][0], num[t][1], sel, h[t],
                    p[f"L{l}_{t}_Wa"], p[f"L{l}_{t}_ba"].reshape(1, HID), pw,
                    wcat[(l + 1, t)], bcat[(l + 1, t)])
                kv[t] = (kvlo, kvhi)
        else:
            for t in NODE_TYPES:
                h[t] = _post_final(
                    num[t][0], num[t][1], sel, h[t],
                    p[f"L{l}_{t}_Wa"], p[f"L{l}_{t}_ba"].reshape(1, HID), pw)
    return h
